# baseline (speedup 1.0000x reference)
import functools

import jax
import jax.numpy as jnp
from jax.experimental import pallas as pl
from jax.experimental.pallas import tpu as pltpu


def _readout_kernel(x_ref, oh_ref, w_ref, r_ref, b_ref, out_ref, wf_ref):
    T = out_ref.shape[0]
    E = out_ref.shape[3]

    @pl.when((pl.program_id(0) == 0) & (pl.program_id(1) == 0))
    def _fold_weights():
        for t in range(T):
            wf_ref[:, t * E:(t + 1) * E] = w_ref[t]

    acc = jnp.dot(x_ref[0], wf_ref[...],
                  preferred_element_type=jnp.float32)
    mT = jnp.transpose(oh_ref[0], (1, 0))
    maskex = jnp.dot(mT, r_ref[...],
                     preferred_element_type=jnp.float32)
    masked = (acc + b_ref[...]) * maskex
    for t in range(T):
        out_ref[t, 0] = masked[:, t * E:(t + 1) * E]


def kernel(output_latents, output_task_index, W, b):
    B, N, D = output_latents.shape
    T, _, E = W.shape

    xb = output_latents.astype(jnp.bfloat16)
    onehot = (output_task_index[:, None, :]
              == jnp.arange(T, dtype=output_task_index.dtype)[None, :, None]
              ).astype(jnp.bfloat16)
    Wb = W.astype(jnp.bfloat16)
    tids = jnp.arange(T * E, dtype=jnp.int32) // E
    R = (jnp.arange(T, dtype=jnp.int32)[:, None] == tids[None, :]).astype(
        jnp.bfloat16)
    bm = b.reshape(1, T * E)

    TN = 512
    grid = (B, N // TN)

    out = pl.pallas_call(
        _readout_kernel,
        grid=grid,
        in_specs=[
            pl.BlockSpec((1, TN, D), lambda b_, n: (b_, n, 0)),
            pl.BlockSpec((1, T, TN), lambda b_, n: (b_, 0, n)),
            pl.BlockSpec((T, D, E), lambda b_, n: (0, 0, 0)),
            pl.BlockSpec((T, T * E), lambda b_, n: (0, 0)),
            pl.BlockSpec((1, T * E), lambda b_, n: (0, 0)),
        ],
        out_specs=pl.BlockSpec((T, 1, TN, E), lambda b_, n: (0, b_, n, 0)),
        out_shape=jax.ShapeDtypeStruct((T, B, N, E), jnp.float32),
        scratch_shapes=[pltpu.VMEM((D, T * E), jnp.bfloat16)],
    )(xb, onehot, Wb, R, bm)
    return out


# speedup vs baseline: 1.2051x; 1.2051x over previous
import functools

import jax
import jax.numpy as jnp
from jax.experimental import pallas as pl
from jax.experimental.pallas import tpu as pltpu


def _readout_kernel(x_ref, oh_ref, w_ref, r_ref, b_ref, out_ref, wf_ref):
    T = out_ref.shape[0]
    E = out_ref.shape[3]

    @pl.when((pl.program_id(0) == 0) & (pl.program_id(1) == 0))
    def _fold_weights():
        for t in range(T):
            wf_ref[:, t * E:(t + 1) * E] = w_ref[t]

    acc = jnp.dot(x_ref[0], wf_ref[...],
                  preferred_element_type=jnp.float32)
    mT = jnp.transpose(oh_ref[0], (1, 0))
    maskex = jnp.dot(mT, r_ref[...],
                     preferred_element_type=jnp.float32)
    masked = (acc + b_ref[...]) * maskex
    for t in range(T):
        out_ref[t, 0] = masked[:, t * E:(t + 1) * E]


def kernel(output_latents, output_task_index, W, b):
    B, N, D = output_latents.shape
    T, _, E = W.shape

    xb = output_latents.astype(jnp.bfloat16)
    onehot = (output_task_index[:, None, :]
              == jnp.arange(T, dtype=output_task_index.dtype)[None, :, None]
              ).astype(jnp.bfloat16)
    Wb = W.astype(jnp.bfloat16)
    tids = jnp.arange(T * E, dtype=jnp.int32) // E
    R = (jnp.arange(T, dtype=jnp.int32)[:, None] == tids[None, :]).astype(
        jnp.bfloat16)
    bm = b.reshape(1, T * E)

    TN = 512
    grid = (B, N // TN)

    out = pl.pallas_call(
        _readout_kernel,
        grid=grid,
        in_specs=[
            pl.BlockSpec((1, TN, D), lambda b_, n: (b_, n, 0)),
            pl.BlockSpec((1, T, TN), lambda b_, n: (b_, 0, n)),
            pl.BlockSpec((T, D, E), lambda b_, n: (0, 0, 0)),
            pl.BlockSpec((T, T * E), lambda b_, n: (0, 0)),
            pl.BlockSpec((1, T * E), lambda b_, n: (0, 0)),
        ],
        out_specs=pl.BlockSpec((T, 1, TN, E), lambda b_, n: (0, b_, n, 0)),
        out_shape=jax.ShapeDtypeStruct((T, B, N, E), jnp.float32),
        scratch_shapes=[pltpu.VMEM((D, T * E), jnp.bfloat16)],
        compiler_params=pltpu.CompilerParams(
            allow_input_fusion=[True, True, True, True, True]),
    )(xb, onehot, Wb, R, bm)
    return out


# speedup vs baseline: 2.7539x; 2.2852x over previous
import jax
import jax.numpy as jnp
from jax.experimental import pallas as pl
from jax.experimental.pallas import tpu as pltpu


def _readout_kernel(x_ref, oh_ref, w_ref, rt_ref, bt_ref, out_ref, wft_ref):
    T = out_ref.shape[0]
    E = out_ref.shape[2]

    @pl.when((pl.program_id(0) == 0) & (pl.program_id(1) == 0))
    def _fold_weights():
        for t in range(T):
            wft_ref[t * E:(t + 1) * E, :] = jnp.transpose(
                w_ref[t], (1, 0)).astype(jnp.bfloat16)

    accT = jax.lax.dot_general(
        wft_ref[...], x_ref[0],
        dimension_numbers=(((1,), (1,)), ((), ())),
        preferred_element_type=jnp.float32)
    maskexT = jnp.dot(rt_ref[...], oh_ref[0],
                      preferred_element_type=jnp.float32)
    maskedT = (accT + bt_ref[...]) * maskexT
    for t in range(T):
        out_ref[t, 0] = maskedT[t * E:(t + 1) * E, :]


def kernel(output_latents, output_task_index, W, b):
    B, N, D = output_latents.shape
    T, _, E = W.shape

    xb = output_latents.astype(jnp.bfloat16)
    onehot = (output_task_index[:, None, :]
              == jnp.arange(T, dtype=output_task_index.dtype)[None, :, None]
              ).astype(jnp.bfloat16)
    tids = jnp.arange(T * E, dtype=jnp.int32) // E
    RT = (tids[:, None] == jnp.arange(T, dtype=jnp.int32)[None, :]).astype(
        jnp.bfloat16)
    bT = b.reshape(T * E, 1)

    TN = 512
    grid = (B, N // TN)

    out = pl.pallas_call(
        _readout_kernel,
        grid=grid,
        in_specs=[
            pl.BlockSpec((1, TN, D), lambda b_, n: (b_, n, 0)),
            pl.BlockSpec((1, T, TN), lambda b_, n: (b_, 0, n)),
            pl.BlockSpec((T, D, E), lambda b_, n: (0, 0, 0)),
            pl.BlockSpec((T * E, T), lambda b_, n: (0, 0)),
            pl.BlockSpec((T * E, 1), lambda b_, n: (0, 0)),
        ],
        out_specs=pl.BlockSpec((T, 1, E, TN), lambda b_, n: (0, b_, 0, n)),
        out_shape=jax.ShapeDtypeStruct((T, B, E, N), jnp.float32),
        scratch_shapes=[pltpu.VMEM((T * E, D), jnp.bfloat16)],
        compiler_params=pltpu.CompilerParams(
            allow_input_fusion=[True, True, True, True, True]),
    )(xb, onehot, W, RT, bT)
    return jnp.transpose(out, (0, 1, 3, 2))


# speedup vs baseline: 3.3732x; 1.2249x over previous
import jax
import jax.numpy as jnp
from jax.experimental import pallas as pl
from jax.experimental.pallas import tpu as pltpu


def _readout_kernel(x_ref, oh_ref, w_ref, rt_ref, bt_ref, out_ref, wft_ref):
    T = out_ref.shape[0]
    E = out_ref.shape[2]

    @pl.when((pl.program_id(0) == 0) & (pl.program_id(1) == 0))
    def _fold_weights():
        for t in range(T):
            wft_ref[t * E:(t + 1) * E, :] = jnp.transpose(
                w_ref[t], (1, 0)).astype(jnp.bfloat16)

    accT = jax.lax.dot_general(
        wft_ref[...], x_ref[0],
        dimension_numbers=(((1,), (1,)), ((), ())),
        preferred_element_type=jnp.float32)
    maskexT = jnp.dot(rt_ref[...], oh_ref[0],
                      preferred_element_type=jnp.float32)
    maskedT = (accT + bt_ref[...]) * maskexT
    for t in range(T):
        out_ref[t, 0] = maskedT[t * E:(t + 1) * E, :]


def kernel(output_latents, output_task_index, W, b):
    B, N, D = output_latents.shape
    T, _, E = W.shape

    xb = output_latents.astype(jnp.bfloat16)
    onehot = (output_task_index[:, None, :]
              == jnp.arange(T, dtype=output_task_index.dtype)[None, :, None]
              ).astype(jnp.bfloat16)
    tids = jnp.arange(T * E, dtype=jnp.int32) // E
    RT = (tids[:, None] == jnp.arange(T, dtype=jnp.int32)[None, :]).astype(
        jnp.bfloat16)
    bT = b.reshape(T * E, 1)

    TN = 1024
    grid = (B, N // TN)

    out = pl.pallas_call(
        _readout_kernel,
        grid=grid,
        in_specs=[
            pl.BlockSpec((1, TN, D), lambda b_, n: (b_, n, 0)),
            pl.BlockSpec((1, T, TN), lambda b_, n: (b_, 0, n)),
            pl.BlockSpec((T, D, E), lambda b_, n: (0, 0, 0)),
            pl.BlockSpec((T * E, T), lambda b_, n: (0, 0)),
            pl.BlockSpec((T * E, 1), lambda b_, n: (0, 0)),
        ],
        out_specs=pl.BlockSpec((T, 1, E, TN), lambda b_, n: (0, b_, 0, n)),
        out_shape=jax.ShapeDtypeStruct((T, B, E, N), jnp.float32),
        scratch_shapes=[pltpu.VMEM((T * E, D), jnp.bfloat16)],
        compiler_params=pltpu.CompilerParams(
            allow_input_fusion=[True, True, True, True, True]),
    )(xb, onehot, W, RT, bT)
    return jnp.transpose(out, (0, 1, 3, 2))
